# Initial kernel scaffold; baseline (speedup 1.0000x reference)
import jax
import jax.numpy as jnp
from jax.experimental import pallas as pl


def kernel(X, group_ids, res_per_gf, W, b):
    raise NotImplementedError("write your pallas kernel here")


# speedup vs baseline: 2.5563x; 2.5563x over previous
import functools

import jax
import jax.numpy as jnp
from jax import lax
from jax.experimental import pallas as pl
from jax.experimental.pallas import tpu as pltpu
from jax.experimental.pallas import tpu_sc as plsc

N = 262144
F = 64
RANK = 17
RANK_PAD = 24
NUM_RF = 16

_NC = 2
_NS = 16
_NW = _NC * _NS
_RPW = N // _NW
_CHUNK = 2048
_NBUF = 2


def _gather_body(idx_hbm, table_hbm, out_hbm, idx_v, rows_v, sems):
    wid = lax.axis_index("s") * _NC + lax.axis_index("c")
    base = wid * _RPW
    nch = _RPW // _CHUNK

    pltpu.sync_copy(idx_hbm.at[pl.ds(base, _RPW)], idx_v)

    copies = [None] * _NBUF
    for ch in range(nch):
        buf = ch % _NBUF
        cp = pltpu.make_async_copy(
            table_hbm.at[idx_v.at[pl.ds(ch * _CHUNK, _CHUNK)]],
            rows_v.at[buf],
            sems.at[buf],
        )
        cp.start()
        copies[buf] = cp
        if ch > 0:
            pbuf = (ch - 1) % _NBUF
            copies[pbuf].wait()
            pltpu.sync_copy(rows_v.at[pbuf],
                            out_hbm.at[pl.ds(base + (ch - 1) * _CHUNK, _CHUNK)])
    last = nch - 1
    copies[last % _NBUF].wait()
    pltpu.sync_copy(rows_v.at[last % _NBUF],
                    out_hbm.at[pl.ds(base + last * _CHUNK, _CHUNK)])


@functools.partial(jax.jit)
def _sc_gather(group_ids, table):
    mesh = plsc.VectorSubcoreMesh(core_axis_name="c", subcore_axis_name="s")
    return pl.kernel(
        _gather_body,
        out_type=jax.ShapeDtypeStruct((N, RANK_PAD), jnp.float32),
        mesh=mesh,
        compiler_params=pltpu.CompilerParams(use_tc_tiling_on_sc=False),
        scratch_types=[
            pltpu.VMEM((_RPW,), jnp.int32),
            pltpu.VMEM((_NBUF, _CHUNK, RANK_PAD), jnp.float32),
            pltpu.SemaphoreType.DMA((_NBUF,)),
        ],
    )(group_ids, table)


_BLK = 2048


def _dense_body(x_ref, bt_ref, w_ref, b_ref, o_ref):
    x = x_ref[...]
    bt = bt_ref[...]
    w = w_ref[...]
    yf = jnp.sum(x * w, axis=1)
    yr = bt[:, 0] + jnp.sum(x[:, :NUM_RF] * bt[:, 1:RANK], axis=1)
    o_ref[...] = yf + yr + b_ref[0]


@jax.jit
def _tc_dense(X, betas, W, b):
    grid = (N // _BLK,)
    return pl.pallas_call(
        _dense_body,
        out_shape=jax.ShapeDtypeStruct((N,), jnp.float32),
        grid=grid,
        in_specs=[
            pl.BlockSpec((_BLK, F), lambda i: (i, 0)),
            pl.BlockSpec((_BLK, RANK_PAD), lambda i: (i, 0)),
            pl.BlockSpec((1, F), lambda i: (0, 0)),
            pl.BlockSpec(memory_space=pltpu.SMEM),
        ],
        out_specs=pl.BlockSpec((_BLK,), lambda i: (i,)),
    )(X, betas, W, b)


def kernel(X, group_ids, res_per_gf, W, b):
    table = jnp.pad(res_per_gf, ((0, 0), (0, RANK_PAD - RANK)))
    betas = _sc_gather(group_ids.astype(jnp.int32), table)
    return _tc_dense(X, betas, W, b)
